# Initial kernel scaffold; baseline (speedup 1.0000x reference)
import jax
import jax.numpy as jnp
from jax.experimental import pallas as pl


def kernel(rois, cls_scores, bbox_deltas, gt_boxes, gt_clses, device):
    raise NotImplementedError("write your pallas kernel here")


# speedup vs baseline: 2.8699x; 2.8699x over previous
import jax
import jax.numpy as jnp
from jax.experimental import pallas as pl

_NEGATIVE = -2
_UPPER = 0.4
_LOWER = 0.1
_NCLS = 80
_BACKGROUND = _NCLS
_TL = 1000


def _loss_kernel(cls_ref, bd_ref, rois_ref, gtt_ref, gtc_ref, acc_ref):
    t = pl.program_id(1)
    cls = cls_ref[0]
    bd = bd_ref[0]
    rois = rois_ref[0]
    gtt = gtt_ref[0]
    gtc = gtc_ref[0]

    tl, C = cls.shape
    M = gtc.shape[1]

    lane_c = jax.lax.broadcasted_iota(jnp.int32, (tl, C), 1)
    rowmax = jnp.max(cls, axis=1, keepdims=True)
    idx = jnp.min(jnp.where(cls == rowmax, lane_c, C), axis=1, keepdims=True)

    logz = rowmax + jnp.log(jnp.sum(jnp.exp(cls - rowmax), axis=1, keepdims=True))

    D = bd.shape[1]
    lane_d = jax.lax.broadcasted_iota(jnp.int32, (tl, D), 1)
    cls_hit = jax.lax.shift_right_logical(lane_d, 2) == idx
    sub = jnp.bitwise_and(lane_d, 3)
    pred = []
    for k in range(4):
        mk = cls_hit & (sub == k)
        sk = jnp.sum(jnp.where(mk, bd, 0.0), axis=1, keepdims=True)
        pred.append(rois[:, k:k + 1] + sk)
    px1, py1, px2, py2 = pred

    gx1, gy1, gx2, gy2 = (gtt[k:k + 1, :] for k in range(4))
    area_a = (px2 - px1) * (py2 - py1)
    area_b = (gx2 - gx1) * (gy2 - gy1)
    iw = jnp.maximum(jnp.minimum(px2, gx2) - jnp.maximum(px1, gx1), 0.0)
    ih = jnp.maximum(jnp.minimum(py2, gy2) - jnp.maximum(py1, gy1), 0.0)
    inter = iw * ih
    iou = inter / (area_a + area_b - inter + 1e-9)
    max_iou = jnp.max(iou, axis=1, keepdims=True)
    lane_m = jax.lax.broadcasted_iota(jnp.int32, (tl, M), 1)
    arg = jnp.min(jnp.where(iou == max_iou, lane_m, M), axis=1, keepdims=True)

    pos = max_iou >= _UPPER
    neg = max_iou < _LOWER
    onehot = lane_m == arg
    pos_label = jnp.sum(jnp.where(onehot, gtc, 0.0), axis=1, keepdims=True)
    label = jnp.where(pos, pos_label, float(_BACKGROUND))

    logit_at = jnp.sum(
        jnp.where(lane_c.astype(jnp.float32) == label, cls, 0.0),
        axis=1, keepdims=True)
    ce = logz - logit_at
    w = (pos | neg).astype(jnp.float32)

    bl = jnp.zeros((tl, 1), jnp.float32)
    for k in range(4):
        gk = jnp.sum(jnp.where(onehot, gtt[k:k + 1, :], 0.0), axis=1, keepdims=True)
        d = pred[k] - gk
        ad = jnp.abs(d)
        bl = bl + jnp.where(ad < 1.0, 0.5 * d * d, ad - 0.5)
    pw = pos.astype(jnp.float32)

    sums = (jnp.sum(ce * w), jnp.sum(w), jnp.sum(pw),
            jnp.sum(neg.astype(jnp.float32)), jnp.sum(bl * pw))
    lane_o = jax.lax.broadcasted_iota(jnp.int32, (1, 128), 1)
    vec = jnp.zeros((1, 128), jnp.float32)
    for j, sv in enumerate(sums):
        vec = vec + jnp.where(lane_o == j, sv, 0.0)

    @pl.when(t == 0)
    def _init():
        acc_ref[0] = vec

    @pl.when(t != 0)
    def _acc():
        acc_ref[0] = acc_ref[0] + vec


def kernel(rois, cls_scores, bbox_deltas, gt_boxes, gt_clses, device):
    N, L, C = cls_scores.shape
    M = gt_boxes.shape[2]
    gtt = jnp.swapaxes(gt_boxes[:, 0], 1, 2)
    gtc = gt_clses.astype(jnp.float32).reshape(N, 1, M)
    T = L // _TL
    acc = pl.pallas_call(
        _loss_kernel,
        grid=(N, T),
        in_specs=[
            pl.BlockSpec((1, _TL, C), lambda n, t: (n, t, 0)),
            pl.BlockSpec((1, _TL, 4 * C), lambda n, t: (n, t, 0)),
            pl.BlockSpec((1, _TL, 4), lambda n, t: (n, t, 0)),
            pl.BlockSpec((1, 4, M), lambda n, t: (n, 0, 0)),
            pl.BlockSpec((1, 1, M), lambda n, t: (n, 0, 0)),
        ],
        out_specs=pl.BlockSpec((1, 1, 128), lambda n, t: (n, 0, 0)),
        out_shape=jax.ShapeDtypeStruct((N, 1, 128), jnp.float32),
    )(cls_scores, bbox_deltas, rois, gtt, gtc)
    acc = acc[:, 0, :]
    s_ce_w, s_w, s_pos, s_neg, s_bl = (acc[:, j] for j in range(5))
    cls_loss = jnp.sum(s_ce_w / jnp.maximum(s_w, 1.0))
    bbox_loss = jnp.sum(jnp.where(s_pos > 0, s_bl / N, 0.0))
    return (cls_loss, bbox_loss, jnp.sum(s_pos), jnp.sum(s_neg))
